# baseline (speedup 1.0000x reference)
import functools

import jax
import jax.numpy as jnp
from jax import lax
from jax.experimental import pallas as pl
from jax.experimental.pallas import tpu as pltpu
from jax.experimental.pallas import tpu_sc as plsc

N_NODES = 10000
N_EDGES = 160000
IN_FEATS = 256
HALF = 128
NSUB = 16
EPT = N_EDGES // NSUB
CH = 128
NCH = 80
EPAD = NCH * CH
NPASS = 4
NSEG = 5120
GARB = NSEG
NPAD = 2 * NSEG
NPT = NSEG // NSUB
NBUF = 2


def _sc_body(x2_hbm, col2_hbm, rowp_hbm, za_hbm, ones_hbm, out4_hbm,
             col_v, row_v, rows_v, agg_sh, sem):
    c = lax.axis_index("c")
    s = lax.axis_index("s")
    nbase = s * NPT

    pltpu.sync_copy(col2_hbm.at[c, s], col_v)

    def pass_body(p, carry):
        feat = p < 2
        half = lax.rem(p, 2)
        pltpu.sync_copy(rowp_hbm.at[half, s], row_v)
        pltpu.sync_copy(za_hbm, agg_sh.at[pl.ds(nbase, NPT)])

        @pl.when(jnp.logical_not(feat))
        def _():
            pltpu.sync_copy(ones_hbm, rows_v.at[0])

        plsc.subcore_barrier()

        lo = jnp.where(feat, 0, (NCH // 2) * c)
        hi = jnp.where(feat, NCH, (NCH // 2) * (c + 1))
        n_iter = hi - lo + jnp.where(feat, NBUF, 0)

        def chunk_body(t, carry2):
            j = lo + t
            jj = jnp.where(feat, j - NBUF, j)

            @pl.when(jj >= lo)
            def _():
                bm = jnp.where(feat, lax.rem(jj, NBUF), 0)

                @pl.when(feat)
                def _():
                    pltpu.make_async_copy(
                        x2_hbm.at[col_v.at[jj]], rows_v.at[bm], sem).wait()

                pltpu.sync_copy(
                    rows_v.at[bm], agg_sh.at[row_v.at[jj]], add=True)

            @pl.when(jnp.logical_and(feat, j < hi))
            def _():
                pltpu.async_copy(
                    x2_hbm.at[col_v.at[j]], rows_v.at[lax.rem(j, NBUF)], sem)

            return carry2

        lax.fori_loop(0, n_iter, chunk_body, 0)
        plsc.subcore_barrier()

        slot = jnp.where(feat, c, 2 + c)
        obase = half * NSEG + nbase
        pltpu.sync_copy(agg_sh.at[pl.ds(nbase, NPT)],
                        out4_hbm.at[slot, pl.ds(obase, NPT)])
        plsc.subcore_barrier()
        return carry

    lax.fori_loop(0, NPASS, pass_body, 0)


@functools.cache
def _make_sc_agg():
  return pl.kernel(
    _sc_body,
    out_type=jax.ShapeDtypeStruct((4, NPAD, HALF), jnp.float32),
    mesh=plsc.VectorSubcoreMesh(core_axis_name="c", subcore_axis_name="s"),
    scratch_types=[
        pltpu.VMEM((NCH, CH), jnp.int32),
        pltpu.VMEM((NCH, CH), jnp.int32),
        pltpu.VMEM((NBUF, CH, HALF), jnp.float32),
        pltpu.VMEM_SHARED((NSEG + 8, HALF), jnp.float32),
        pltpu.SemaphoreType.DMA,
    ],
  )


BM = 1000


def _tc_body(x_ref, agg_ref, w_ref, b_ref, o_ref):
    deg = agg_ref[2, :, 0:1] + agg_ref[3, :, 0:1]
    inv = 1.0 / jnp.maximum(deg, 1.0)
    h = jnp.concatenate(
        [x_ref[...], agg_ref[0] * inv, agg_ref[1] * inv], axis=1)
    o_ref[...] = lax.dot_general(
        h, w_ref[...], (((1,), (1,)), ((), ())),
        preferred_element_type=jnp.float32) + b_ref[...]


def _tc_matmul(x, out4, W, b):
    return pl.pallas_call(
        _tc_body,
        grid=(N_NODES // BM,),
        in_specs=[
            pl.BlockSpec((BM, IN_FEATS), lambda i: (i, 0)),
            pl.BlockSpec((4, BM, HALF), lambda i: (0, i, 0)),
            pl.BlockSpec((IN_FEATS, 2 * IN_FEATS), lambda i: (0, 0)),
            pl.BlockSpec((1, IN_FEATS), lambda i: (0, 0)),
        ],
        out_specs=pl.BlockSpec((BM, IN_FEATS), lambda i: (i, 0)),
        out_shape=jax.ShapeDtypeStruct((N_NODES, IN_FEATS), jnp.float32),
    )(x, out4, W, b.reshape(1, IN_FEATS))


def kernel(x, edge_index, W, b):
    row = edge_index[0].astype(jnp.int32)
    col = edge_index[1].astype(jnp.int32)
    x2 = x.reshape(2 * N_NODES, HALF)
    pad_c = jnp.zeros((NSUB, EPAD - EPT), jnp.int32)
    pad_r = jnp.full((NSUB, EPAD - EPT), GARB, jnp.int32)
    colt = col.reshape(NSUB, EPT)
    col2 = jnp.stack([
        jnp.concatenate([2 * colt, pad_c], axis=1),
        jnp.concatenate([2 * colt + 1, pad_c], axis=1),
    ]).reshape(2, NSUB, NCH, CH)
    rowt = row.reshape(NSUB, EPT)
    rowp = jnp.stack([
        jnp.concatenate([jnp.where(rowt < NSEG, rowt, GARB), pad_r], axis=1),
        jnp.concatenate([jnp.where(rowt >= NSEG, rowt - NSEG, GARB), pad_r],
                        axis=1),
    ]).reshape(2, NSUB, NCH, CH)
    za = jnp.zeros((NPT, HALF), jnp.float32)
    ones = jnp.ones((CH, HALF), jnp.float32)
    out4 = _make_sc_agg()(x2, col2, rowp, za, ones)
    return _tc_matmul(x, out4, W, b)


# speedup vs baseline: 1.8056x; 1.8056x over previous
import functools

import jax
import jax.numpy as jnp
from jax import lax
from jax.experimental import pallas as pl
from jax.experimental.pallas import tpu as pltpu
from jax.experimental.pallas import tpu_sc as plsc

N_NODES = 10000
N_EDGES = 160000
IN_FEATS = 256
HALF = 128
NSUB = 16
EPT = N_EDGES // NSUB
CH = 128
NCH = 80
EPAD = NCH * CH
NPASS = 2
NSEG = 10240
GARB = NSEG
NPAD = NSEG
NPT = NSEG // NSUB


def _sc_body(x2_hbm, col2_hbm, rowp_hbm, za_hbm, ones_hbm, out4_hbm,
             col_v, row_v, rows_v, agg_sh, sem):
    c = lax.axis_index("c")
    s = lax.axis_index("s")
    nbase = s * NPT

    pltpu.sync_copy(col2_hbm.at[c, s], col_v)

    def pass_body(p, carry):
        pltpu.sync_copy(rowp_hbm.at[0, s], row_v)
        pltpu.sync_copy(za_hbm, agg_sh.at[pl.ds(nbase, NPT)])

        @pl.when(p >= 1)
        def _():
            pltpu.sync_copy(ones_hbm, rows_v)

        plsc.subcore_barrier()

        def chunk_body(j, carry2):
            @pl.when(p < 1)
            def _():
                pltpu.async_copy(x2_hbm.at[col_v.at[j]], rows_v, sem).wait()

            pltpu.sync_copy(rows_v, agg_sh.at[row_v.at[j]], add=True)
            return carry2

        lo = jnp.where(p < 1, 0, (NCH // 2) * c)
        hi = jnp.where(p < 1, NCH, (NCH // 2) * (c + 1))
        lax.fori_loop(lo, hi, chunk_body, 0)
        plsc.subcore_barrier()

        slot = jnp.where(p < 1, c, 2 + c)
        obase = nbase
        pltpu.sync_copy(agg_sh.at[pl.ds(nbase, NPT)],
                        out4_hbm.at[slot, pl.ds(obase, NPT)])
        plsc.subcore_barrier()
        return carry

    lax.fori_loop(0, NPASS, pass_body, 0)


@functools.cache
def _make_sc_agg():
  return pl.kernel(
    _sc_body,
    out_type=jax.ShapeDtypeStruct((4, NPAD, HALF), jnp.float32),
    mesh=plsc.VectorSubcoreMesh(core_axis_name="c", subcore_axis_name="s"),
    scratch_types=[
        pltpu.VMEM((NCH, CH), jnp.int32),
        pltpu.VMEM((NCH, CH), jnp.int32),
        pltpu.VMEM((CH, HALF), jnp.float32),
        pltpu.VMEM_SHARED((NSEG + 8, HALF), jnp.float32),
        pltpu.SemaphoreType.DMA,
    ],
  )


BM = 1000


def _tc_body(x_ref, agg_ref, w_ref, b_ref, o_ref):
    deg = agg_ref[2, :, 0:1] + agg_ref[3, :, 0:1]
    inv = 1.0 / jnp.maximum(deg, 1.0)
    h = jnp.concatenate(
        [x_ref[...], agg_ref[0] * inv, agg_ref[1] * inv], axis=1)
    o_ref[...] = lax.dot_general(
        h, w_ref[...], (((1,), (1,)), ((), ())),
        preferred_element_type=jnp.float32) + b_ref[...]


def _tc_matmul(x, out4, W, b):
    return pl.pallas_call(
        _tc_body,
        grid=(N_NODES // BM,),
        in_specs=[
            pl.BlockSpec((BM, IN_FEATS), lambda i: (i, 0)),
            pl.BlockSpec((4, BM, HALF), lambda i: (0, i, 0)),
            pl.BlockSpec((IN_FEATS, 2 * IN_FEATS), lambda i: (0, 0)),
            pl.BlockSpec((1, IN_FEATS), lambda i: (0, 0)),
        ],
        out_specs=pl.BlockSpec((BM, IN_FEATS), lambda i: (i, 0)),
        out_shape=jax.ShapeDtypeStruct((N_NODES, IN_FEATS), jnp.float32),
    )(x, out4, W, b.reshape(1, IN_FEATS))


def kernel(x, edge_index, W, b):
    row = edge_index[0].astype(jnp.int32)
    col = edge_index[1].astype(jnp.int32)
    x2 = x.reshape(2 * N_NODES, HALF)
    pad_c = jnp.zeros((NSUB, EPAD - EPT), jnp.int32)
    pad_r = jnp.full((NSUB, EPAD - EPT), GARB, jnp.int32)
    colt = col.reshape(NSUB, EPT)
    col2 = jnp.stack([
        jnp.concatenate([2 * colt, pad_c], axis=1),
        jnp.concatenate([2 * colt + 1, pad_c], axis=1),
    ]).reshape(2, NSUB, NCH, CH)
    rowt = row.reshape(NSUB, EPT)
    rowp = jnp.concatenate([rowt, pad_r], axis=1).reshape(1, NSUB, NCH, CH)
    za = jnp.zeros((NPT, HALF), jnp.float32)
    ones = jnp.ones((CH, HALF), jnp.float32)
    out4 = _make_sc_agg()(x2, col2, rowp, za, ones)
    return _tc_matmul(x, out4, W, b)
